# Initial kernel scaffold; baseline (speedup 1.0000x reference)
import jax
import jax.numpy as jnp
from jax.experimental import pallas as pl


def kernel(x_vals, x_cons, x_econs, x_obj, ei_c2v, ea_c2v, ei_v2c, ea_v2c, ei_e2v, ea_e2v, ei_v2e, ea_v2e, ei_v2o, ea_v2o, ei_o2v, ea_o2v, ei_c2o, ea_c2o, ei_o2c, ea_o2c, ei_e2o, ea_e2o, ei_o2e, ea_o2e, encW1, encb1, encW2, encb2, convW1, convb1, convW2, convb2, predW1, predb1, predW2, predb2):
    raise NotImplementedError("write your pallas kernel here")


# speedup vs baseline: 1.1692x; 1.1692x over previous
import functools

import jax
import jax.numpy as jnp
from jax import lax
from jax.experimental import pallas as pl
from jax.experimental.pallas import tpu as pltpu
from jax.experimental.pallas import tpu_sc as plsc

_NV, _NC, _NE, _NO = 50000, 25000, 25000, 1000
_CH = 128

_REL_DEFS = [
    ('c2v', _NC, _NV, 500000), ('v2c', _NV, _NC, 500000),
    ('e2v', _NE, _NV, 500000), ('v2e', _NV, _NE, 500000),
    ('v2o', _NV, _NO, 100000), ('o2v', _NO, _NV, 100000),
    ('c2o', _NC, _NO, 50000), ('o2c', _NO, _NC, 50000),
    ('e2o', _NE, _NO, 50000), ('o2e', _NO, _NE, 50000),
]
_REL_J = {name: j for j, (name, _, _, _) in enumerate(_REL_DEFS)}
_SRC_RELS = {'v': ['v2c', 'v2e', 'v2o'], 'c': ['c2v', 'c2o'],
             'e': ['e2v', 'e2o'], 'o': ['o2v', 'o2c', 'o2e']}
_DST_RELS = {'v': ['c2v', 'e2v', 'o2v'], 'c': ['v2c', 'o2c'],
             'e': ['v2e', 'o2e'], 'o': ['v2o', 'c2o', 'e2o']}
_N_T = {'v': _NV, 'c': _NC, 'e': _NE, 'o': _NO}
_LAYER_SEQ = [0, 1, 0, 1]


def _ceil_to(x, m):
    return (x + m - 1) // m * m


def _pad_dst(n_dst):
    return _ceil_to(n_dst + 16, 2048)


def _zrows_of(rpt16):
    if rpt16 <= 1024:
        return rpt16
    z = rpt16 // -(-rpt16 // 1024)
    assert rpt16 % z == 0
    return z


_NDP_MAX = _pad_dst(_NV)
_ZROWS = {51200 // 16: 800, 26624 // 16: 416, 2048 // 16: 128}

_CFG = []
_EOFF, _ZOFF = {}, {}
_e_acc = _z_acc = 0
for _name, _ns, _nd, _E in _REL_DEFS:
    _ndp = _pad_dst(_nd)
    _Ep = _ceil_to(_E, 32 * _CH)
    _CFG.append((_name, _ns, _nd, _ndp, _Ep))
    _EOFF[_name] = _e_acc
    _ZOFF[_name] = _z_acc
    _e_acc += _Ep
    _z_acc += _ndp
_E_TOT, _Z_TOT = _e_acc, _z_acc


def _build_round_kernel():
    mesh = plsc.VectorSubcoreMesh(core_axis_name="c", subcore_axis_name="s")

    out_type = [jax.ShapeDtypeStruct((4, _Z_TOT, 16), jnp.float32),
                jax.ShapeDtypeStruct((4, _Z_TOT, 16), jnp.float32)]
    scratch_types = [
        pltpu.VMEM((16,), jnp.int32),
        pltpu.VMEM((_CH,), jnp.int32),
        pltpu.VMEM((_CH,), jnp.int32),
        pltpu.VMEM((_CH,), jnp.float32),
        pltpu.VMEM((_CH, 16), jnp.float32),
        pltpu.VMEM((_CH, 16), jnp.float32),
        pltpu.VMEM((800, 16), jnp.float32),
        pltpu.VMEM_SHARED((_NDP_MAX, 16), jnp.float32),
        pltpu.SemaphoreType.DMA,
    ]

    @functools.partial(
        pl.kernel, mesh=mesh,
        compiler_params=pltpu.CompilerParams(use_tc_tiling_on_sc=False),
        out_type=out_type, scratch_types=scratch_types,
    )
    def k(flag_h, src_h, dst_h, ea_h, y0, y1, y2_, y3, y4, y5, y6, y7, y8, y9,
          z_all, d_all, flg, sidx, didx, eab, rows, ones, zbuf, shared, sem):
        y_hs = [y0, y1, y2_, y3, y4, y5, y6, y7, y8, y9]
        c = lax.axis_index("c")
        s = lax.axis_index("s")
        z16 = jnp.zeros((16,), jnp.float32)
        o16 = jnp.ones((16,), jnp.float32)

        def zb(i, carry):
            zbuf[i, 0:16] = z16
            ones[i % _CH, 0:16] = o16
            return carry
        lax.fori_loop(0, 800, zb, 0)
        pltpu.sync_copy(flag_h, flg)
        fv = flg[0:16]

        def zero_phase(ndp):
            rpt16 = ndp // 16
            zrows = _ZROWS[rpt16]
            base = s * rpt16
            for t in range(rpt16 // zrows):
                pltpu.sync_copy(zbuf.at[pl.ds(0, zrows)],
                                shared.at[pl.ds(base + t * zrows, zrows)])
            plsc.subcore_barrier()

        def flush_phase(ndp, zoff, out_h, q):
            plsc.subcore_barrier()
            rpt16 = ndp // 16
            base = s * rpt16
            pltpu.sync_copy(shared.at[pl.ds(base, rpt16)],
                            out_h.at[q].at[pl.ds(zoff + base, rpt16)])
            plsc.subcore_barrier()

        for r, (name, ns, nd, ndp, E_p) in enumerate(_CFG):
            y_h = y_hs[r]
            eoff = _EOFF[name]
            cpt = E_p // _CH // 16

            def qpass(qq, qcarry):
                q = c * 2 + qq
                zero_phase(ndp)

                def chunk(j, carry):
                    off = eoff + (s * cpt + j) * _CH
                    pltpu.sync_copy(src_h.at[pl.ds(off, _CH)], sidx)
                    pltpu.sync_copy(ea_h.at[pl.ds(off, _CH)], eab)
                    pltpu.async_copy(y_h.at[q].at[sidx], rows, sem).wait()

                    def sc_body(g, cc):
                        av = eab[pl.ds(g * 16, 16)]
                        for l in range(16):
                            a = av[l]
                            e = g * 16 + l
                            rows[e, 0:16] = rows[e, 0:16] * a
                        return cc
                    lax.fori_loop(0, _CH // 16, sc_body, 0)
                    pltpu.sync_copy(dst_h.at[pl.ds(off, _CH)], didx)
                    pltpu.sync_copy(rows, shared.at[didx], add=True)
                    return carry
                lax.fori_loop(0, cpt, chunk, 0)
                flush_phase(ndp, _ZOFF[name], z_all, q)
                return qcarry
            lax.fori_loop(0, 2, qpass, 0)

        @pl.when(fv[0] == 1)
        def _deg():
            for r, (name, ns, nd, ndp, E_p) in enumerate(_CFG):
                eoff = _EOFF[name]
                cpt = E_p // _CH // 32
                w = c * 16 + s
                zero_phase(ndp)

                def chunk(j, carry):
                    off = eoff + (w * cpt + j) * _CH
                    pltpu.sync_copy(dst_h.at[pl.ds(off, _CH)], didx)
                    pltpu.sync_copy(ones, shared.at[didx], add=True)
                    return carry
                lax.fori_loop(0, cpt, chunk, 0)
                flush_phase(ndp, _ZOFF[name], d_all, c * 2)

    return k


_BLK = 1024


def _enc_proj_call(x, We1, be1, We2, be2, W1s, n):
    kk = W1s.shape[0]
    grid = (pl.cdiv(n, _BLK),)

    def body(x_r, We1_r, be1_r, We2_r, be2_r, W1s_r, h_r, *y_rs):
        h = jnp.maximum(
            jnp.dot(x_r[...], We1_r[...], preferred_element_type=jnp.float32)
            + be1_r[...], 0.0)
        h = jnp.dot(h, We2_r[...], preferred_element_type=jnp.float32) + be2_r[...]
        h_r[...] = h
        for j in range(kk):
            yj = jnp.dot(h, W1s_r[j], preferred_element_type=jnp.float32)
            for q in range(4):
                y_rs[j][q] = yj[:, 16 * q:16 * (q + 1)]

    outs = ([jax.ShapeDtypeStruct((n, 128), jnp.float32)]
            + [jax.ShapeDtypeStruct((4, n, 16), jnp.float32)] * kk)
    in_specs = [
        pl.BlockSpec((_BLK, 16), lambda r: (r, 0)),
        pl.BlockSpec((16, 64), lambda r: (0, 0)),
        pl.BlockSpec((1, 64), lambda r: (0, 0)),
        pl.BlockSpec((64, 128), lambda r: (0, 0)),
        pl.BlockSpec((1, 128), lambda r: (0, 0)),
        pl.BlockSpec((kk, 128, 64), lambda r: (0, 0, 0)),
    ]
    out_specs = ([pl.BlockSpec((_BLK, 128), lambda r: (r, 0))]
                 + [pl.BlockSpec((4, _BLK, 16), lambda r: (0, r, 0))] * kk)
    return pl.pallas_call(body, grid=grid, in_specs=in_specs,
                          out_specs=out_specs, out_shape=outs)(
        x, We1, be1, We2, be2, W1s)


def _post_call(z_all, d_all, rels, x_old, W2s, b1s, b2s, W1n, mode, n):
    kk = len(rels)
    m = 0 if W1n is None else W1n.shape[0]
    grid = (pl.cdiv(n, _BLK),)

    def body(*refs):
        z_rs = refs[0:kk]
        d_rs = refs[kk:2 * kk]
        x_r = refs[2 * kk]
        W2_r, b1_r, b2_r = refs[2 * kk + 1:2 * kk + 4]
        pos = 2 * kk + 4
        W1n_r = refs[pos] if m else None
        pos += 1 if m else 0
        xn_r, h2_r = refs[pos], refs[pos + 1]
        y_rs = refs[pos + 2:]
        os_ = []
        for j in range(kk):
            z = jnp.concatenate([z_rs[j][0], z_rs[j][1],
                                 z_rs[j][2], z_rs[j][3]], axis=1)
            deg = d_rs[j][0][:, 0:1] + d_rs[j][2][:, 0:1]
            h = z / (deg + 1.0)
            o = jnp.dot(jnp.maximum(h + b1_r[j], 0.0), W2_r[j],
                        preferred_element_type=jnp.float32) + b2_r[j]
            os_.append(o)
        if mode in ('v', 'o'):
            h2 = jnp.concatenate([os_[0], 0.5 * (os_[1] + os_[2])], axis=1)
        else:
            h2 = jnp.concatenate([os_[0], os_[1]], axis=1)
        xn = 0.5 * (jnp.maximum(h2, 0.0) + x_r[...])
        h2_r[...] = h2
        xn_r[...] = xn
        for j in range(m):
            yj = jnp.dot(xn, W1n_r[j], preferred_element_type=jnp.float32)
            for q in range(4):
                y_rs[j][q] = yj[:, 16 * q:16 * (q + 1)]

    zoffb = [_ZOFF[rl] // _BLK for rl in rels]
    in_specs = ([pl.BlockSpec((4, _BLK, 16),
                              functools.partial(lambda o, r: (0, o + r, 0), o))
                 for o in zoffb] * 2
                + [pl.BlockSpec((_BLK, 128), lambda r: (r, 0)),
                   pl.BlockSpec((kk, 64, 64), lambda r: (0, 0, 0)),
                   pl.BlockSpec((kk, 1, 64), lambda r: (0, 0, 0)),
                   pl.BlockSpec((kk, 1, 64), lambda r: (0, 0, 0))])
    args = [z_all] * kk + [d_all] * kk + [x_old, W2s, b1s, b2s]
    if m:
        in_specs.append(pl.BlockSpec((m, 128, 64), lambda r: (0, 0, 0)))
        args.append(W1n)
    outs = ([jax.ShapeDtypeStruct((n, 128), jnp.float32)] * 2
            + [jax.ShapeDtypeStruct((4, n, 16), jnp.float32)] * m)
    out_specs = ([pl.BlockSpec((_BLK, 128), lambda r: (r, 0))] * 2
                 + [pl.BlockSpec((4, _BLK, 16), lambda r: (0, r, 0))] * m)
    return pl.pallas_call(body, grid=grid, in_specs=in_specs,
                          out_specs=out_specs, out_shape=outs)(*args)


def _pred_call(h0, h1, W1, b1, W2, b2, relu_out, n):
    grid = (pl.cdiv(n, _BLK),)

    def body(h0_r, h1_r, W1_r, b1_r, W2_r, b2_r, o_r):
        cols = []
        for t, h_r in enumerate((h0_r, h1_r)):
            a = jnp.maximum(
                jnp.dot(h_r[...], W1_r[t], preferred_element_type=jnp.float32)
                + b1_r[t], 0.0)
            cols.append(jnp.dot(a, W2_r[t],
                                preferred_element_type=jnp.float32) + b2_r[t])
        o = jnp.concatenate(cols, axis=1)
        if relu_out:
            o = jnp.maximum(o, 0.0)
        o_r[...] = o

    in_specs = [
        pl.BlockSpec((_BLK, 128), lambda r: (r, 0)),
        pl.BlockSpec((_BLK, 128), lambda r: (r, 0)),
        pl.BlockSpec((2, 128, 64), lambda r: (0, 0, 0)),
        pl.BlockSpec((2, 1, 64), lambda r: (0, 0, 0)),
        pl.BlockSpec((2, 64, 1), lambda r: (0, 0, 0)),
        pl.BlockSpec((2, 1, 1), lambda r: (0, 0, 0)),
    ]
    return pl.pallas_call(
        body, grid=grid, in_specs=in_specs,
        out_specs=pl.BlockSpec((_BLK, 2), lambda r: (r, 0)),
        out_shape=jax.ShapeDtypeStruct((n, 2), jnp.float32))(
        h0, h1, W1, b1, W2, b2)


def kernel(x_vals, x_cons, x_econs, x_obj,
           ei_c2v, ea_c2v, ei_v2c, ea_v2c, ei_e2v, ea_e2v, ei_v2e, ea_v2e,
           ei_v2o, ea_v2o, ei_o2v, ea_o2v, ei_c2o, ea_c2o, ei_o2c, ea_o2c,
           ei_e2o, ea_e2o, ei_o2e, ea_o2e,
           encW1, encb1, encW2, encb2,
           convW1, convb1, convW2, convb2,
           predW1, predb1, predW2, predb2):
    ei = {'c2v': ei_c2v, 'v2c': ei_v2c, 'e2v': ei_e2v, 'v2e': ei_v2e,
          'v2o': ei_v2o, 'o2v': ei_o2v, 'c2o': ei_c2o, 'o2c': ei_o2c,
          'e2o': ei_e2o, 'o2e': ei_o2e}
    ea = {'c2v': ea_c2v, 'v2c': ea_v2c, 'e2v': ea_e2v, 'v2e': ea_v2e,
          'v2o': ea_v2o, 'o2v': ea_o2v, 'c2o': ea_c2o, 'o2c': ea_o2c,
          'e2o': ea_e2o, 'o2e': ea_o2e}
    x0 = {'v': x_vals, 'c': x_cons, 'e': x_econs, 'o': x_obj}
    enc_i = {'v': 0, 'c': 1, 'e': 2, 'o': 3}

    s_parts, d_parts, a_parts = [], [], []
    for name, ns, nd, ndp, E_p in _CFG:
        E = ei[name].shape[1]
        pn = E_p - E
        sink = nd + (jnp.arange(pn, dtype=jnp.int32) % 16)
        s_parts.append(jnp.concatenate([ei[name][0],
                                        jnp.zeros((pn,), jnp.int32)]))
        d_parts.append(jnp.concatenate([ei[name][1], sink]))
        a_parts.append(jnp.concatenate([ea[name][:, 0],
                                        jnp.zeros((pn,), jnp.float32)]))
    src_all = jnp.concatenate(s_parts)
    dst_all = jnp.concatenate(d_parts)
    ea_all = jnp.concatenate(a_parts)
    rk = _build_round_kernel()

    xs, ys, h2s = {}, {}, {}
    for t in ('v', 'c', 'e', 'o'):
        W1s = jnp.stack([convW1[_LAYER_SEQ[0], _REL_J[r]]
                         for r in _SRC_RELS[t]])
        ti = enc_i[t]
        outs = _enc_proj_call(x0[t], encW1[ti], encb1[ti][None, :],
                              encW2[ti], encb2[ti][None, :], W1s, _N_T[t])
        xs[t] = outs[0]
        for j, r in enumerate(_SRC_RELS[t]):
            ys[r] = outs[1 + j]

    h2_snap = {}
    d_all = None
    for rnd in range(4):
        i = _LAYER_SEQ[rnd]
        flag = jnp.full((16,), 1 if rnd == 0 else 0, jnp.int32)
        args = [flag, src_all, dst_all, ea_all]
        args += [ys[name] for name, _, _, _, _ in _CFG]
        z_all, d_new = rk(*args)
        if rnd == 0:
            d_all = d_new
        ys = {}
        for t in ('v', 'c', 'e', 'o'):
            rels = _DST_RELS[t]
            W2s = jnp.stack([convW2[i, _REL_J[r]] for r in rels])
            b1s = jnp.stack([convb1[i, _REL_J[r]][None, :] for r in rels])
            b2s = jnp.stack([convb2[i, _REL_J[r]][None, :] for r in rels])
            if rnd < 3:
                i_nx = _LAYER_SEQ[rnd + 1]
                W1n = jnp.stack([convW1[i_nx, _REL_J[r]]
                                 for r in _SRC_RELS[t]])
            else:
                W1n = None
            res = _post_call(z_all, d_all, rels,
                             xs[t], W2s, b1s, b2s, W1n, t, _N_T[t])
            xs[t], h2s[t] = res[0], res[1]
            if rnd < 3:
                for j, r in enumerate(_SRC_RELS[t]):
                    ys[r] = res[2 + j]
        if rnd in (1, 3):
            st = 0 if rnd == 1 else 1
            for t in ('c', 'e', 'v'):
                h2_snap[(st, t)] = h2s[t]

    def pred(ti, t, relu_out, n):
        return _pred_call(h2_snap[(0, t)], h2_snap[(1, t)],
                          predW1[ti], predb1[ti][:, None, :],
                          predW2[ti], predb2[ti][:, None, :], relu_out, n)

    vals = pred(0, 'v', True, _NV)
    cons = pred(1, 'c', False, _NC)
    econs = pred(2, 'e', False, _NE)
    return (vals, cons, econs)


# speedup vs baseline: 2.7570x; 2.3579x over previous
import functools

import jax
import jax.numpy as jnp
from jax import lax
from jax.experimental import pallas as pl
from jax.experimental.pallas import tpu as pltpu
from jax.experimental.pallas import tpu_sc as plsc

_NV, _NC, _NE, _NO = 50000, 25000, 25000, 1000
_CH = 128

_REL_DEFS = [
    ('c2v', _NC, _NV, 500000), ('v2c', _NV, _NC, 500000),
    ('e2v', _NE, _NV, 500000), ('v2e', _NV, _NE, 500000),
    ('v2o', _NV, _NO, 100000), ('o2v', _NO, _NV, 100000),
    ('c2o', _NC, _NO, 50000), ('o2c', _NO, _NC, 50000),
    ('e2o', _NE, _NO, 50000), ('o2e', _NO, _NE, 50000),
]
_REL_J = {name: j for j, (name, _, _, _) in enumerate(_REL_DEFS)}
_SRC_RELS = {'v': ['v2c', 'v2e', 'v2o'], 'c': ['c2v', 'c2o'],
             'e': ['e2v', 'e2o'], 'o': ['o2v', 'o2c', 'o2e']}
_DST_RELS = {'v': ['c2v', 'e2v', 'o2v'], 'c': ['v2c', 'o2c'],
             'e': ['v2e', 'o2e'], 'o': ['v2o', 'c2o', 'e2o']}
_N_T = {'v': _NV, 'c': _NC, 'e': _NE, 'o': _NO}
_LAYER_SEQ = [0, 1, 0, 1]


def _ceil_to(x, m):
    return (x + m - 1) // m * m


def _pad_dst(n_dst):
    return _ceil_to(n_dst + 16, 2048)


def _zrows_of(rpt16):
    if rpt16 <= 1024:
        return rpt16
    z = rpt16 // -(-rpt16 // 1024)
    assert rpt16 % z == 0
    return z


_NDP_MAX = _pad_dst(_NV)
_ZROWS = {51200 // 16: 400, 26624 // 16: 208, 2048 // 16: 128}

_CFG = []
_EOFF, _ZOFF = {}, {}
_e_acc = _z_acc = 0
for _name, _ns, _nd, _E in _REL_DEFS:
    _ndp = _pad_dst(_nd)
    _Ep = _ceil_to(_E, 64 * _CH)
    _CFG.append((_name, _ns, _nd, _ndp, _Ep))
    _EOFF[_name] = _e_acc
    _ZOFF[_name] = _z_acc
    _e_acc += _Ep
    _z_acc += _ndp
_E_TOT, _Z_TOT = _e_acc, _z_acc


def _build_round_kernel():
    mesh = plsc.VectorSubcoreMesh(core_axis_name="c", subcore_axis_name="s")

    _BS_MAX = 124

    out_type = [jax.ShapeDtypeStruct((4, _Z_TOT, 16), jnp.float32),
                jax.ShapeDtypeStruct((4, _Z_TOT, 16), jnp.float32)]
    scratch_types = [
        pltpu.VMEM((16,), jnp.int32),
        pltpu.VMEM((_BS_MAX, _CH), jnp.int32),
        pltpu.VMEM((_BS_MAX, _CH), jnp.int32),
        pltpu.VMEM((_BS_MAX, _CH), jnp.float32),
        pltpu.VMEM((_CH, 16), jnp.float32),
        pltpu.VMEM((_CH, 16), jnp.float32),
        pltpu.VMEM((_CH, 16), jnp.float32),
        pltpu.VMEM((_CH, 16), jnp.float32),
        pltpu.VMEM((_CH, 16), jnp.float32),
        pltpu.VMEM((400, 16), jnp.float32),
        pltpu.VMEM_SHARED((_NDP_MAX, 16), jnp.float32),
        pltpu.SemaphoreType.DMA, pltpu.SemaphoreType.DMA,
        pltpu.SemaphoreType.DMA, pltpu.SemaphoreType.DMA,
        pltpu.SemaphoreType.DMA, pltpu.SemaphoreType.DMA,
        pltpu.SemaphoreType.DMA, pltpu.SemaphoreType.DMA,
    ]

    @functools.partial(
        pl.kernel, mesh=mesh,
        compiler_params=pltpu.CompilerParams(use_tc_tiling_on_sc=False),
        out_type=out_type, scratch_types=scratch_types,
    )
    def k(flag_h, src_h, dst_h, ea_h, y0, y1, y2_, y3, y4, y5, y6, y7, y8, y9,
          z_all, d_all, flg, sia, dia, eaa, r0, r1, r2, r3, ones, zbuf,
          shared, sg0, sg1, sg2, sg3, ss0, ss1, ss2, ss3):
        y_hs = [y0, y1, y2_, y3, y4, y5, y6, y7, y8, y9]
        rows = [r0, r1, r2, r3]
        sg = [sg0, sg1, sg2, sg3]
        ss = [ss0, ss1, ss2, ss3]
        c = lax.axis_index("c")
        s = lax.axis_index("s")
        z16 = jnp.zeros((16,), jnp.float32)
        o16 = jnp.ones((16,), jnp.float32)

        def zb(i, carry):
            zbuf[i % 400, 0:16] = z16
            ones[i % _CH, 0:16] = o16
            return carry
        lax.fori_loop(0, 400, zb, 0)
        pltpu.sync_copy(flag_h, flg)
        fv = flg[0:16]

        def zero_phase(ndp):
            rpt16 = ndp // 16
            zrows = _ZROWS[rpt16]
            base = s * rpt16
            for t in range(rpt16 // zrows):
                pltpu.sync_copy(zbuf.at[pl.ds(0, zrows)],
                                shared.at[pl.ds(base + t * zrows, zrows)])
            plsc.subcore_barrier()

        def flush_phase(ndp, zoff, out_h, q):
            plsc.subcore_barrier()
            rpt16 = ndp // 16
            base = s * rpt16
            pltpu.sync_copy(shared.at[pl.ds(base, rpt16)],
                            out_h.at[q].at[pl.ds(zoff + base, rpt16)])
            plsc.subcore_barrier()

        def g_issue(y_h, q, j, b):
            return pltpu.async_copy(y_h.at[q].at[sia.at[j]], rows[b], sg[b])

        def g_wait(y_h, q, j, b):
            pltpu.make_async_copy(y_h.at[q].at[sia.at[j]], rows[b],
                                  sg[b]).wait()

        def s_issue(j, b):
            return pltpu.async_copy(rows[b], shared.at[dia.at[j]], ss[b],
                                    add=True)

        def s_wait(j, b):
            pltpu.make_async_copy(rows[b], shared.at[dia.at[j]],
                                  ss[b]).wait()

        for r, (name, ns, nd, ndp, E_p) in enumerate(_CFG):
            y_h = y_hs[r]
            erow = _EOFF[name] // _CH
            cpt = E_p // _CH // 16

            nblk = -(-cpt // _BS_MAX)
            bs = cpt // nblk
            assert bs * nblk == cpt and bs % 4 == 0

            def qpass(qq, qcarry):
                q = c * 2 + qq
                zero_phase(ndp)
                for blk in range(nblk):
                    tb = erow + s * cpt + blk * bs
                    pltpu.sync_copy(src_h.at[pl.ds(tb, bs)],
                                    sia.at[pl.ds(0, bs)])
                    pltpu.sync_copy(dst_h.at[pl.ds(tb, bs)],
                                    dia.at[pl.ds(0, bs)])
                    pltpu.sync_copy(ea_h.at[pl.ds(tb, bs)],
                                    eaa.at[pl.ds(0, bs)])
                    g_issue(y_h, q, 0, 0)
                    g_issue(y_h, q, 1, 1)

                    def quad(jj, carry):
                        for b in range(4):
                            j = 4 * jj + b
                            b2 = (b + 2) % 4

                            @pl.when(j >= 2)
                            def _():
                                s_wait(j - 2, b2)

                            @pl.when(j + 2 < bs)
                            def _():
                                g_issue(y_h, q, j + 2, b2)
                            g_wait(y_h, q, j, b)

                            def sc_body(g, cc):
                                av = eaa[j, pl.ds(g * 16, 16)]
                                rb = rows[b]
                                for l in range(16):
                                    a = av[l]
                                    e = g * 16 + l
                                    rb[e, 0:16] = rb[e, 0:16] * a
                                return cc
                            lax.fori_loop(0, _CH // 16, sc_body, 0)
                            s_issue(j, b)
                        return carry
                    lax.fori_loop(0, bs // 4, quad, 0)
                    s_wait(bs - 2, 2)
                    s_wait(bs - 1, 3)
                flush_phase(ndp, _ZOFF[name], z_all, q)
                return qcarry
            lax.fori_loop(0, 2, qpass, 0)

        @pl.when(fv[0] == 1)
        def _deg():
            for r, (name, ns, nd, ndp, E_p) in enumerate(_CFG):
                erow = _EOFF[name] // _CH
                cpt = E_p // _CH // 32
                w = c * 16 + s
                zero_phase(ndp)
                pltpu.sync_copy(dst_h.at[pl.ds(erow + w * cpt, cpt)],
                                dia.at[pl.ds(0, cpt)])

                def d_issue(j, b):
                    return pltpu.async_copy(ones, shared.at[dia.at[j]],
                                            ss[b], add=True)

                def d_wait(j, b):
                    pltpu.make_async_copy(ones, shared.at[dia.at[j]],
                                          ss[b]).wait()

                def pair(jj, carry):
                    for b in range(2):
                        j = 2 * jj + b

                        @pl.when(j >= 2)
                        def _():
                            d_wait(j - 2, b)
                        d_issue(j, b)
                    return carry
                lax.fori_loop(0, cpt // 2, pair, 0)
                d_wait(cpt - 2, 0)
                d_wait(cpt - 1, 1)
                flush_phase(ndp, _ZOFF[name], d_all, c * 2)

    return k


_BLK = 1024


def _enc_proj_call(x, We1, be1, We2, be2, W1s, n):
    kk = W1s.shape[0]
    grid = (pl.cdiv(n, _BLK),)

    def body(x_r, We1_r, be1_r, We2_r, be2_r, W1s_r, h_r, *y_rs):
        h = jnp.maximum(
            jnp.dot(x_r[...], We1_r[...], preferred_element_type=jnp.float32)
            + be1_r[...], 0.0)
        h = jnp.dot(h, We2_r[...], preferred_element_type=jnp.float32) + be2_r[...]
        h_r[...] = h
        for j in range(kk):
            yj = jnp.dot(h, W1s_r[j], preferred_element_type=jnp.float32)
            for q in range(4):
                y_rs[j][q] = yj[:, 16 * q:16 * (q + 1)]

    outs = ([jax.ShapeDtypeStruct((n, 128), jnp.float32)]
            + [jax.ShapeDtypeStruct((4, n, 16), jnp.float32)] * kk)
    in_specs = [
        pl.BlockSpec((_BLK, 16), lambda r: (r, 0)),
        pl.BlockSpec((16, 64), lambda r: (0, 0)),
        pl.BlockSpec((1, 64), lambda r: (0, 0)),
        pl.BlockSpec((64, 128), lambda r: (0, 0)),
        pl.BlockSpec((1, 128), lambda r: (0, 0)),
        pl.BlockSpec((kk, 128, 64), lambda r: (0, 0, 0)),
    ]
    out_specs = ([pl.BlockSpec((_BLK, 128), lambda r: (r, 0))]
                 + [pl.BlockSpec((4, _BLK, 16), lambda r: (0, r, 0))] * kk)
    return pl.pallas_call(body, grid=grid, in_specs=in_specs,
                          out_specs=out_specs, out_shape=outs)(
        x, We1, be1, We2, be2, W1s)


def _post_call(z_all, d_all, rels, x_old, W2s, b1s, b2s, W1n, mode, n):
    kk = len(rels)
    m = 0 if W1n is None else W1n.shape[0]
    grid = (pl.cdiv(n, _BLK),)

    def body(*refs):
        z_rs = refs[0:kk]
        d_rs = refs[kk:2 * kk]
        x_r = refs[2 * kk]
        W2_r, b1_r, b2_r = refs[2 * kk + 1:2 * kk + 4]
        pos = 2 * kk + 4
        W1n_r = refs[pos] if m else None
        pos += 1 if m else 0
        xn_r, h2_r = refs[pos], refs[pos + 1]
        y_rs = refs[pos + 2:]
        os_ = []
        for j in range(kk):
            z = jnp.concatenate([z_rs[j][0], z_rs[j][1],
                                 z_rs[j][2], z_rs[j][3]], axis=1)
            deg = d_rs[j][0][:, 0:1] + d_rs[j][2][:, 0:1]
            h = z / (deg + 1.0)
            o = jnp.dot(jnp.maximum(h + b1_r[j], 0.0), W2_r[j],
                        preferred_element_type=jnp.float32) + b2_r[j]
            os_.append(o)
        if mode in ('v', 'o'):
            h2 = jnp.concatenate([os_[0], 0.5 * (os_[1] + os_[2])], axis=1)
        else:
            h2 = jnp.concatenate([os_[0], os_[1]], axis=1)
        xn = 0.5 * (jnp.maximum(h2, 0.0) + x_r[...])
        h2_r[...] = h2
        xn_r[...] = xn
        for j in range(m):
            yj = jnp.dot(xn, W1n_r[j], preferred_element_type=jnp.float32)
            for q in range(4):
                y_rs[j][q] = yj[:, 16 * q:16 * (q + 1)]

    zoffb = [_ZOFF[rl] // _BLK for rl in rels]
    in_specs = ([pl.BlockSpec((4, _BLK, 16),
                              functools.partial(lambda o, r: (0, o + r, 0), o))
                 for o in zoffb] * 2
                + [pl.BlockSpec((_BLK, 128), lambda r: (r, 0)),
                   pl.BlockSpec((kk, 64, 64), lambda r: (0, 0, 0)),
                   pl.BlockSpec((kk, 1, 64), lambda r: (0, 0, 0)),
                   pl.BlockSpec((kk, 1, 64), lambda r: (0, 0, 0))])
    args = [z_all] * kk + [d_all] * kk + [x_old, W2s, b1s, b2s]
    if m:
        in_specs.append(pl.BlockSpec((m, 128, 64), lambda r: (0, 0, 0)))
        args.append(W1n)
    outs = ([jax.ShapeDtypeStruct((n, 128), jnp.float32)] * 2
            + [jax.ShapeDtypeStruct((4, n, 16), jnp.float32)] * m)
    out_specs = ([pl.BlockSpec((_BLK, 128), lambda r: (r, 0))] * 2
                 + [pl.BlockSpec((4, _BLK, 16), lambda r: (0, r, 0))] * m)
    return pl.pallas_call(body, grid=grid, in_specs=in_specs,
                          out_specs=out_specs, out_shape=outs)(*args)


def _pred_call(h0, h1, W1, b1, W2, b2, relu_out, n):
    grid = (pl.cdiv(n, _BLK),)

    def body(h0_r, h1_r, W1_r, b1_r, W2_r, b2_r, o_r):
        cols = []
        for t, h_r in enumerate((h0_r, h1_r)):
            a = jnp.maximum(
                jnp.dot(h_r[...], W1_r[t], preferred_element_type=jnp.float32)
                + b1_r[t], 0.0)
            cols.append(jnp.dot(a, W2_r[t],
                                preferred_element_type=jnp.float32) + b2_r[t])
        o = jnp.concatenate(cols, axis=1)
        if relu_out:
            o = jnp.maximum(o, 0.0)
        o_r[...] = o

    in_specs = [
        pl.BlockSpec((_BLK, 128), lambda r: (r, 0)),
        pl.BlockSpec((_BLK, 128), lambda r: (r, 0)),
        pl.BlockSpec((2, 128, 64), lambda r: (0, 0, 0)),
        pl.BlockSpec((2, 1, 64), lambda r: (0, 0, 0)),
        pl.BlockSpec((2, 64, 1), lambda r: (0, 0, 0)),
        pl.BlockSpec((2, 1, 1), lambda r: (0, 0, 0)),
    ]
    return pl.pallas_call(
        body, grid=grid, in_specs=in_specs,
        out_specs=pl.BlockSpec((_BLK, 2), lambda r: (r, 0)),
        out_shape=jax.ShapeDtypeStruct((n, 2), jnp.float32))(
        h0, h1, W1, b1, W2, b2)


def kernel(x_vals, x_cons, x_econs, x_obj,
           ei_c2v, ea_c2v, ei_v2c, ea_v2c, ei_e2v, ea_e2v, ei_v2e, ea_v2e,
           ei_v2o, ea_v2o, ei_o2v, ea_o2v, ei_c2o, ea_c2o, ei_o2c, ea_o2c,
           ei_e2o, ea_e2o, ei_o2e, ea_o2e,
           encW1, encb1, encW2, encb2,
           convW1, convb1, convW2, convb2,
           predW1, predb1, predW2, predb2):
    ei = {'c2v': ei_c2v, 'v2c': ei_v2c, 'e2v': ei_e2v, 'v2e': ei_v2e,
          'v2o': ei_v2o, 'o2v': ei_o2v, 'c2o': ei_c2o, 'o2c': ei_o2c,
          'e2o': ei_e2o, 'o2e': ei_o2e}
    ea = {'c2v': ea_c2v, 'v2c': ea_v2c, 'e2v': ea_e2v, 'v2e': ea_v2e,
          'v2o': ea_v2o, 'o2v': ea_o2v, 'c2o': ea_c2o, 'o2c': ea_o2c,
          'e2o': ea_e2o, 'o2e': ea_o2e}
    x0 = {'v': x_vals, 'c': x_cons, 'e': x_econs, 'o': x_obj}
    enc_i = {'v': 0, 'c': 1, 'e': 2, 'o': 3}

    s_parts, d_parts, a_parts = [], [], []
    for name, ns, nd, ndp, E_p in _CFG:
        E = ei[name].shape[1]
        pn = E_p - E
        sink = nd + (jnp.arange(pn, dtype=jnp.int32) % 16)
        s_parts.append(jnp.concatenate([ei[name][0],
                                        jnp.zeros((pn,), jnp.int32)]))
        d_parts.append(jnp.concatenate([ei[name][1], sink]))
        a_parts.append(jnp.concatenate([ea[name][:, 0],
                                        jnp.zeros((pn,), jnp.float32)]))
    src_all = jnp.concatenate(s_parts).reshape(-1, _CH)
    dst_all = jnp.concatenate(d_parts).reshape(-1, _CH)
    ea_all = jnp.concatenate(a_parts).reshape(-1, _CH)
    rk = _build_round_kernel()

    xs, ys, h2s = {}, {}, {}
    for t in ('v', 'c', 'e', 'o'):
        W1s = jnp.stack([convW1[_LAYER_SEQ[0], _REL_J[r]]
                         for r in _SRC_RELS[t]])
        ti = enc_i[t]
        outs = _enc_proj_call(x0[t], encW1[ti], encb1[ti][None, :],
                              encW2[ti], encb2[ti][None, :], W1s, _N_T[t])
        xs[t] = outs[0]
        for j, r in enumerate(_SRC_RELS[t]):
            ys[r] = outs[1 + j]

    h2_snap = {}
    d_all = None
    for rnd in range(4):
        i = _LAYER_SEQ[rnd]
        flag = jnp.full((16,), 1 if rnd == 0 else 0, jnp.int32)
        args = [flag, src_all, dst_all, ea_all]
        args += [ys[name] for name, _, _, _, _ in _CFG]
        z_all, d_new = rk(*args)
        if rnd == 0:
            d_all = d_new
        ys = {}
        for t in ('v', 'c', 'e', 'o'):
            rels = _DST_RELS[t]
            W2s = jnp.stack([convW2[i, _REL_J[r]] for r in rels])
            b1s = jnp.stack([convb1[i, _REL_J[r]][None, :] for r in rels])
            b2s = jnp.stack([convb2[i, _REL_J[r]][None, :] for r in rels])
            if rnd < 3:
                i_nx = _LAYER_SEQ[rnd + 1]
                W1n = jnp.stack([convW1[i_nx, _REL_J[r]]
                                 for r in _SRC_RELS[t]])
            else:
                W1n = None
            res = _post_call(z_all, d_all, rels,
                             xs[t], W2s, b1s, b2s, W1n, t, _N_T[t])
            xs[t], h2s[t] = res[0], res[1]
            if rnd < 3:
                for j, r in enumerate(_SRC_RELS[t]):
                    ys[r] = res[2 + j]
        if rnd in (1, 3):
            st = 0 if rnd == 1 else 1
            for t in ('c', 'e', 'v'):
                h2_snap[(st, t)] = h2s[t]

    def pred(ti, t, relu_out, n):
        return _pred_call(h2_snap[(0, t)], h2_snap[(1, t)],
                          predW1[ti], predb1[ti][:, None, :],
                          predW2[ti], predb2[ti][:, None, :], relu_out, n)

    vals = pred(0, 'v', True, _NV)
    cons = pred(1, 'c', False, _NC)
    econs = pred(2, 'e', False, _NE)
    return (vals, cons, econs)
